# Initial kernel scaffold; baseline (speedup 1.0000x reference)
import jax
import jax.numpy as jnp
from jax.experimental import pallas as pl


def kernel(x, edge_index_l0, edge_index_l1, W_l0, b_l0, W_r0, W_l1, b_l1, W_r1, W_post, b_post):
    raise NotImplementedError("write your pallas kernel here")


# speedup vs baseline: 5.3542x; 5.3542x over previous
import functools

import jax
import jax.numpy as jnp
from jax import lax
from jax.experimental import pallas as pl
from jax.experimental.pallas import tpu as pltpu
from jax.experimental.pallas import tpu_sc as plsc

N = 50000
N1 = 16000
N2 = 4096
E1 = 400000
E2 = 102400
D = 128
DH = D // 2

NC = 2
NS = 16
NW = NC * NS

LANES = 16
CHUNK = 128

E1_PAD = 401408
T0 = E1_PAD // NS
K0 = T0 // CHUNK
ACC0 = 16128
SENT0 = N1

T1 = E2 // NW
K1 = T1 // CHUNK
ACC1 = 4096

_MESH = plsc.VectorSubcoreMesh(
    core_axis_name="c", subcore_axis_name="s", num_cores=NC, num_subcores=NS)

_Z16 = functools.partial(jnp.zeros, (LANES,), jnp.float32)
_O16 = functools.partial(jnp.ones, (LANES,), jnp.float32)


def _fill_zero_rows(ref, n_rows, n_cols):
    z = _Z16()
    def body(i, carry):
        for j in range(n_cols // LANES):
            ref[i, pl.ds(j * LANES, LANES)] = z
        return carry
    lax.fori_loop(0, n_rows, body, 0)


def _fill_vec(ref, n, val16):
    def body(i, carry):
        ref[pl.ds(i * LANES, LANES)] = val16
        return carry
    lax.fori_loop(0, n // LANES, body, 0)


def _zero_shared_rows(zsrc, acc_sh, base_row, slice_rows):
    r0 = 0
    while r0 < slice_rows:
        nrows = min(CHUNK, slice_rows - r0)
        pltpu.sync_copy(zsrc.at[pl.ds(0, nrows)],
                        acc_sh.at[pl.ds(base_row + r0, nrows)])
        r0 += nrows


def _copy_out_rows(acc_sh, out_ref, base_row, out_row, slice_rows):
    r0 = 0
    while r0 < slice_rows:
        nrows = min(CHUNK, slice_rows - r0)
        pltpu.sync_copy(acc_sh.at[pl.ds(base_row + r0, nrows)],
                        out_ref.at[pl.ds(out_row + r0, nrows)])
        r0 += nrows


@functools.partial(
    pl.kernel,
    out_type=(
        jax.ShapeDtypeStruct((NC * ACC0, DH), jnp.float32),
        jax.ShapeDtypeStruct((1, ACC0), jnp.float32),
    ),
    mesh=_MESH,
    compiler_params=pltpu.CompilerParams(use_tc_tiling_on_sc=False),
    scratch_types=[
        pltpu.VMEM((CHUNK,), jnp.int32),
        pltpu.VMEM((CHUNK,), jnp.int32),
        pltpu.VMEM((CHUNK,), jnp.int32),
        pltpu.VMEM((CHUNK, DH), jnp.float32),
        pltpu.VMEM((CHUNK,), jnp.float32),
        pltpu.VMEM((ACC0 // NS,), jnp.float32),
        pltpu.VMEM_SHARED((ACC0, DH), jnp.float32),
        pltpu.VMEM_SHARED((ACC0,), jnp.float32),
        pltpu.SemaphoreType.DMA,
    ],
    name="sc_sage_agg0",
)
def _sc_agg0(x2_hbm, src_hbm, dst_hbm, out_sum, out_cnt,
             src_v, gidx_v, dst_v, rows_v, ones_v, zcnt_v, acc_sh, cnt_sh,
             sem):
    c = lax.axis_index("c")
    s = lax.axis_index("s")

    _fill_zero_rows(rows_v, CHUNK, DH)
    _fill_vec(ones_v, CHUNK, _O16())
    _fill_vec(zcnt_v, ACC0 // NS, _Z16())

    slice_rows = ACC0 // NS
    base_row = s * slice_rows
    _zero_shared_rows(rows_v, acc_sh, base_row, slice_rows)
    pltpu.sync_copy(zcnt_v, cnt_sh.at[pl.ds(s * slice_rows, slice_rows)])

    plsc.subcore_barrier()

    edge_base = s * T0

    def chunk(kk, carry):
        off = edge_base + kk * CHUNK
        pltpu.sync_copy(src_hbm.at[pl.ds(off, CHUNK)], src_v)
        pltpu.sync_copy(dst_hbm.at[pl.ds(off, CHUNK)], dst_v)
        for j in range(CHUNK // LANES):
            sl = pl.ds(j * LANES, LANES)
            gidx_v[sl] = src_v[sl] * 2 + c
        pltpu.async_copy(x2_hbm.at[gidx_v], rows_v, sem).wait()
        pltpu.sync_copy(rows_v, acc_sh.at[dst_v], add=True)

        @pl.when(c == 0)
        def _():
            pltpu.sync_copy(ones_v, cnt_sh.at[dst_v], add=True)
        return carry

    lax.fori_loop(0, K0, chunk, 0)

    plsc.subcore_barrier()

    _copy_out_rows(acc_sh, out_sum, base_row, c * ACC0 + base_row, slice_rows)

    @pl.when(jnp.logical_and(c == 0, s == 0))
    def _():
        pltpu.sync_copy(cnt_sh, out_cnt.at[0])


@functools.partial(
    pl.kernel,
    out_type=(
        jax.ShapeDtypeStruct((NC * ACC1, D), jnp.float32),
        jax.ShapeDtypeStruct((NC, ACC1), jnp.float32),
    ),
    mesh=_MESH,
    scratch_types=[
        pltpu.VMEM((CHUNK,), jnp.int32),
        pltpu.VMEM((CHUNK,), jnp.int32),
        pltpu.VMEM((CHUNK, D), jnp.float32),
        pltpu.VMEM((CHUNK,), jnp.float32),
        pltpu.VMEM((ACC1 // NS,), jnp.float32),
        pltpu.VMEM_SHARED((ACC1, D), jnp.float32),
        pltpu.VMEM_SHARED((ACC1,), jnp.float32),
        pltpu.SemaphoreType.DMA,
    ],
    name="sc_sage_agg1",
)
def _sc_agg1(h_hbm, src_hbm, dst_hbm, out_sum, out_cnt,
             src_v, dst_v, rows_v, ones_v, zcnt_v, acc_sh, cnt_sh, sem):
    c = lax.axis_index("c")
    s = lax.axis_index("s")
    wid = s * NC + c

    _fill_zero_rows(rows_v, CHUNK, D)
    _fill_vec(ones_v, CHUNK, _O16())
    _fill_vec(zcnt_v, ACC1 // NS, _Z16())

    slice_rows = ACC1 // NS
    base_row = s * slice_rows
    _zero_shared_rows(rows_v, acc_sh, base_row, slice_rows)
    pltpu.sync_copy(zcnt_v, cnt_sh.at[pl.ds(s * slice_rows, slice_rows)])

    plsc.subcore_barrier()

    edge_base = wid * T1

    def chunk(kk, carry):
        off = edge_base + kk * CHUNK
        pltpu.sync_copy(src_hbm.at[pl.ds(off, CHUNK)], src_v)
        pltpu.sync_copy(dst_hbm.at[pl.ds(off, CHUNK)], dst_v)
        pltpu.async_copy(h_hbm.at[src_v], rows_v, sem).wait()
        pltpu.sync_copy(rows_v, acc_sh.at[dst_v], add=True)
        pltpu.sync_copy(ones_v, cnt_sh.at[dst_v], add=True)
        return carry

    lax.fori_loop(0, K1, chunk, 0)

    plsc.subcore_barrier()

    _copy_out_rows(acc_sh, out_sum, base_row, c * ACC1 + base_row, slice_rows)

    @pl.when(s == 0)
    def _():
        pltpu.sync_copy(cnt_sh, out_cnt.at[c])


def _tc_layer0_body(sum_ref, cnt_ref, x_ref, wl_ref, bl_ref, wr_ref, out_ref):
    recip = 1.0 / jnp.maximum(cnt_ref[...], 1.0)
    mlo = sum_ref[0] * recip
    mhi = sum_ref[1] * recip
    wl = wl_ref[...]
    h = lax.dot_general(mlo, wl[:, :DH], (((1,), (1,)), ((), ())),
                        preferred_element_type=jnp.float32)
    h = h + lax.dot_general(mhi, wl[:, DH:], (((1,), (1,)), ((), ())),
                            preferred_element_type=jnp.float32)
    h = h + bl_ref[...]
    h = h + lax.dot_general(x_ref[...], wr_ref[...], (((1,), (1,)), ((), ())),
                            preferred_element_type=jnp.float32)
    out_ref[...] = jnp.maximum(h, 0.0)


def _tc_layer0(sums2, cnt_col, x, W_l0, b_l0, W_r0):
    BN = 2000
    return pl.pallas_call(
        _tc_layer0_body,
        grid=(N1 // BN,),
        in_specs=[
            pl.BlockSpec((NC, BN, DH), lambda i: (0, i, 0)),
            pl.BlockSpec((BN, 1), lambda i: (i, 0)),
            pl.BlockSpec((BN, D), lambda i: (i, 0)),
            pl.BlockSpec((D, D), lambda i: (0, 0)),
            pl.BlockSpec((1, D), lambda i: (0, 0)),
            pl.BlockSpec((D, D), lambda i: (0, 0)),
        ],
        out_specs=pl.BlockSpec((BN, D), lambda i: (i, 0)),
        out_shape=jax.ShapeDtypeStruct((N1, D), jnp.float32),
        name="tc_sage_layer0",
    )(sums2, cnt_col, x, W_l0, b_l0, W_r0)


def _tc_layer1_body(sum_ref, cntc_ref, h_ref, wl_ref, bl_ref, wr_ref,
                    wp_ref, bp_ref, out_ref):
    ssum = sum_ref[0] + sum_ref[1]
    cnt = cntc_ref[:, 0:1] + cntc_ref[:, 1:2]
    mean = ssum / jnp.maximum(cnt, 1.0)
    h1 = lax.dot_general(mean, wl_ref[...], (((1,), (1,)), ((), ())),
                         preferred_element_type=jnp.float32)
    h1 = h1 + bl_ref[...]
    h1 = h1 + lax.dot_general(h_ref[...], wr_ref[...], (((1,), (1,)), ((), ())),
                              preferred_element_type=jnp.float32)
    o = lax.dot_general(h1, wp_ref[...], (((1,), (1,)), ((), ())),
                        preferred_element_type=jnp.float32)
    o = o + bp_ref[...]
    m = jnp.max(o, axis=1, keepdims=True)
    lse = jnp.log(jnp.sum(jnp.exp(o - m), axis=1, keepdims=True)) + m
    out_ref[...] = o - lse


def _tc_layer1(sums, cnts_col, h, W_l1, b_l1, W_r1, W_post, b_post):
    return pl.pallas_call(
        _tc_layer1_body,
        grid=(1,),
        in_specs=[
            pl.BlockSpec((NC, N2, D), lambda i: (0, 0, 0)),
            pl.BlockSpec((N2, NC), lambda i: (0, 0)),
            pl.BlockSpec((N2, D), lambda i: (0, 0)),
            pl.BlockSpec((D, D), lambda i: (0, 0)),
            pl.BlockSpec((1, D), lambda i: (0, 0)),
            pl.BlockSpec((D, D), lambda i: (0, 0)),
            pl.BlockSpec((D, D), lambda i: (0, 0)),
            pl.BlockSpec((1, D), lambda i: (0, 0)),
        ],
        out_specs=pl.BlockSpec((N2, D), lambda i: (0, 0)),
        out_shape=jax.ShapeDtypeStruct((N2, D), jnp.float32),
        name="tc_sage_layer1",
    )(sums, cnts_col, h, W_l1, b_l1, W_r1, W_post, b_post)


def kernel(x, edge_index_l0, edge_index_l1, W_l0, b_l0, W_r0,
           W_l1, b_l1, W_r1, W_post, b_post):
    src0 = edge_index_l0[0].astype(jnp.int32)
    dst0 = edge_index_l0[1].astype(jnp.int32)
    src1 = edge_index_l1[0].astype(jnp.int32)
    dst1 = edge_index_l1[1].astype(jnp.int32)

    npad = E1_PAD - E1
    src0 = jnp.concatenate([src0, jnp.zeros((npad,), jnp.int32)])
    dst0 = jnp.concatenate([dst0, jnp.full((npad,), SENT0, jnp.int32)])

    x2 = x.reshape(2 * N, DH)
    sums0, cnt0 = _sc_agg0(x2, src0, dst0)
    sums0 = sums0.reshape(NC, ACC0, DH)
    cnt0_col = cnt0.reshape(ACC0, 1)
    h = _tc_layer0(sums0, cnt0_col, x, W_l0, b_l0.reshape(1, D), W_r0)

    sums1, cnts1 = _sc_agg1(h, src1, dst1)
    sums1 = sums1.reshape(NC, ACC1, D)
    cnts1_col = cnts1.T
    return _tc_layer1(sums1, cnts1_col, h, W_l1, b_l1.reshape(1, D), W_r1,
                      W_post, b_post.reshape(1, D))


# speedup vs baseline: 10.6898x; 1.9965x over previous
import functools

import jax
import jax.numpy as jnp
from jax import lax
from jax.experimental import pallas as pl
from jax.experimental.pallas import tpu as pltpu
from jax.experimental.pallas import tpu_sc as plsc

N = 50000
N1 = 16000
N2 = 4096
E1 = 400000
E2 = 102400
D = 128
DH = D // 2

NC = 2
NS = 16
NW = NC * NS

LANES = 16
CHUNK = 128

E1_PAD = 401408
T0 = E1_PAD // NS
K0 = T0 // CHUNK
ACC0 = 16128
SENT0 = N1

T1 = E2 // NW
K1 = T1 // CHUNK
ACC1 = 4096

_MESH = plsc.VectorSubcoreMesh(
    core_axis_name="c", subcore_axis_name="s", num_cores=NC, num_subcores=NS)

_Z16 = functools.partial(jnp.zeros, (LANES,), jnp.float32)
_O16 = functools.partial(jnp.ones, (LANES,), jnp.float32)


def _fill_zero_rows(ref, n_rows, n_cols):
    z = _Z16()
    def body(i, carry):
        for j in range(n_cols // LANES):
            ref[i, pl.ds(j * LANES, LANES)] = z
        return carry
    lax.fori_loop(0, n_rows, body, 0)


def _fill_vec(ref, n, val16):
    def body(i, carry):
        ref[pl.ds(i * LANES, LANES)] = val16
        return carry
    lax.fori_loop(0, n // LANES, body, 0)


def _zero_shared_rows(zsrc, acc_sh, base_row, slice_rows):
    r0 = 0
    while r0 < slice_rows:
        nrows = min(CHUNK, slice_rows - r0)
        pltpu.sync_copy(zsrc.at[pl.ds(0, nrows)],
                        acc_sh.at[pl.ds(base_row + r0, nrows)])
        r0 += nrows


def _copy_out_rows(acc_sh, out_ref, base_row, out_row, slice_rows):
    r0 = 0
    while r0 < slice_rows:
        nrows = min(CHUNK, slice_rows - r0)
        pltpu.sync_copy(acc_sh.at[pl.ds(base_row + r0, nrows)],
                        out_ref.at[pl.ds(out_row + r0, nrows)])
        r0 += nrows


KH0 = K0 // 2


@functools.partial(
    pl.kernel,
    out_type=(
        jax.ShapeDtypeStruct((NC * ACC0, DH), jnp.float32),
        jax.ShapeDtypeStruct((1, ACC0), jnp.float32),
    ),
    mesh=_MESH,
    compiler_params=pltpu.CompilerParams(use_tc_tiling_on_sc=False),
    scratch_types=[
        pltpu.VMEM((KH0, CHUNK), jnp.int32),
        pltpu.VMEM((KH0, CHUNK), jnp.int32),
        pltpu.VMEM((CHUNK, DH), jnp.float32),
        pltpu.VMEM((CHUNK, DH), jnp.float32),
        pltpu.VMEM((CHUNK,), jnp.float32),
        pltpu.VMEM((ACC0 // NS,), jnp.float32),
        pltpu.VMEM_SHARED((ACC0, DH), jnp.float32),
        pltpu.VMEM_SHARED((ACC0,), jnp.float32),
        pltpu.SemaphoreType.DMA,
        pltpu.SemaphoreType.DMA,
    ],
    name="sc_sage_agg0",
)
def _sc_agg0(x2_hbm, src_hbm, dst_hbm, out_sum, out_cnt,
             gidx_v, dst_v, rba, rbb, ones_v, zcnt_v, acc_sh, cnt_sh,
             sema, semb):
    c = lax.axis_index("c")
    s = lax.axis_index("s")

    _fill_zero_rows(rba, CHUNK, DH)
    _fill_vec(ones_v, CHUNK, _O16())
    _fill_vec(zcnt_v, ACC0 // NS, _Z16())

    slice_rows = ACC0 // NS
    base_row = s * slice_rows
    _zero_shared_rows(rba, acc_sh, base_row, slice_rows)
    pltpu.sync_copy(zcnt_v, cnt_sh.at[pl.ds(s * slice_rows, slice_rows)])

    plsc.subcore_barrier()

    two16 = jnp.full((LANES,), 2, jnp.int32)

    def scatter(kk, rb):
        pltpu.sync_copy(rb, acc_sh.at[dst_v.at[kk]], add=True)

        @pl.when(c == 0)
        def _():
            pltpu.sync_copy(ones_v, cnt_sh.at[dst_v.at[kk]], add=True)

    for half in range(2):
        pltpu.sync_copy(src_hbm.at[s, half], gidx_v)
        pltpu.sync_copy(dst_hbm.at[s, half], dst_v)

        def xform(r, carry):
            for j in range(CHUNK // LANES):
                sl = pl.ds(j * LANES, LANES)
                gidx_v[r, sl] = gidx_v[r, sl] * two16 + c
            return carry
        lax.fori_loop(0, KH0, xform, 0)

        pltpu.async_copy(x2_hbm.at[gidx_v.at[0]], rba, sema)

        def pair(i, carry):
            a = 2 * i
            pltpu.async_copy(x2_hbm.at[gidx_v.at[a + 1]], rbb, semb)
            pltpu.make_async_copy(x2_hbm.at[gidx_v.at[a]], rba, sema).wait()
            scatter(a, rba)
            pltpu.async_copy(x2_hbm.at[gidx_v.at[a + 2]], rba, sema)
            pltpu.make_async_copy(x2_hbm.at[gidx_v.at[a + 1]], rbb, semb).wait()
            scatter(a + 1, rbb)
            return carry
        lax.fori_loop(0, KH0 // 2 - 1, pair, 0)

        last = KH0 - 2
        pltpu.async_copy(x2_hbm.at[gidx_v.at[last + 1]], rbb, semb)
        pltpu.make_async_copy(x2_hbm.at[gidx_v.at[last]], rba, sema).wait()
        scatter(last, rba)
        pltpu.make_async_copy(x2_hbm.at[gidx_v.at[last + 1]], rbb, semb).wait()
        scatter(last + 1, rbb)

    plsc.subcore_barrier()

    _copy_out_rows(acc_sh, out_sum, base_row, c * ACC0 + base_row, slice_rows)

    @pl.when(jnp.logical_and(c == 0, s == 0))
    def _():
        pltpu.sync_copy(cnt_sh, out_cnt.at[0])


@functools.partial(
    pl.kernel,
    out_type=(
        jax.ShapeDtypeStruct((NC * ACC1, D), jnp.float32),
        jax.ShapeDtypeStruct((NC, ACC1), jnp.float32),
    ),
    mesh=_MESH,
    scratch_types=[
        pltpu.VMEM((K1, CHUNK), jnp.int32),
        pltpu.VMEM((K1, CHUNK), jnp.int32),
        pltpu.VMEM((CHUNK, D), jnp.float32),
        pltpu.VMEM((CHUNK, D), jnp.float32),
        pltpu.VMEM((CHUNK,), jnp.float32),
        pltpu.VMEM((ACC1 // NS,), jnp.float32),
        pltpu.VMEM_SHARED((ACC1, D), jnp.float32),
        pltpu.VMEM_SHARED((ACC1,), jnp.float32),
        pltpu.SemaphoreType.DMA,
        pltpu.SemaphoreType.DMA,
    ],
    name="sc_sage_agg1",
)
def _sc_agg1(h_hbm, src_hbm, dst_hbm, out_sum, out_cnt,
             src_v, dst_v, rba, rbb, ones_v, zcnt_v, acc_sh, cnt_sh,
             sema, semb):
    c = lax.axis_index("c")
    s = lax.axis_index("s")
    wid = s * NC + c

    _fill_zero_rows(rba, CHUNK, D)
    _fill_vec(ones_v, CHUNK, _O16())
    _fill_vec(zcnt_v, ACC1 // NS, _Z16())

    slice_rows = ACC1 // NS
    base_row = s * slice_rows
    _zero_shared_rows(rba, acc_sh, base_row, slice_rows)
    pltpu.sync_copy(zcnt_v, cnt_sh.at[pl.ds(s * slice_rows, slice_rows)])

    plsc.subcore_barrier()

    pltpu.sync_copy(src_hbm.at[wid], src_v)
    pltpu.sync_copy(dst_hbm.at[wid], dst_v)

    def scatter(kk, rb):
        pltpu.sync_copy(rb, acc_sh.at[dst_v.at[kk]], add=True)
        pltpu.sync_copy(ones_v, cnt_sh.at[dst_v.at[kk]], add=True)

    pltpu.async_copy(h_hbm.at[src_v.at[0]], rba, sema)

    def pair(i, carry):
        a = 2 * i
        pltpu.async_copy(h_hbm.at[src_v.at[a + 1]], rbb, semb)
        pltpu.make_async_copy(h_hbm.at[src_v.at[a]], rba, sema).wait()
        scatter(a, rba)
        pltpu.async_copy(h_hbm.at[src_v.at[a + 2]], rba, sema)
        pltpu.make_async_copy(h_hbm.at[src_v.at[a + 1]], rbb, semb).wait()
        scatter(a + 1, rbb)
        return carry
    lax.fori_loop(0, (K1 - 1) // 2, pair, 0)

    pltpu.make_async_copy(h_hbm.at[src_v.at[K1 - 1]], rba, sema).wait()
    scatter(K1 - 1, rba)

    plsc.subcore_barrier()

    _copy_out_rows(acc_sh, out_sum, base_row, c * ACC1 + base_row, slice_rows)

    @pl.when(s == 0)
    def _():
        pltpu.sync_copy(cnt_sh, out_cnt.at[c])


def _tc_layer0_body(sum_ref, cnt_ref, x_ref, wl_ref, bl_ref, wr_ref, out_ref):
    recip = 1.0 / jnp.maximum(cnt_ref[...], 1.0)
    mlo = sum_ref[0] * recip
    mhi = sum_ref[1] * recip
    wl = wl_ref[...]
    h = lax.dot_general(mlo, wl[:, :DH], (((1,), (1,)), ((), ())),
                        preferred_element_type=jnp.float32)
    h = h + lax.dot_general(mhi, wl[:, DH:], (((1,), (1,)), ((), ())),
                            preferred_element_type=jnp.float32)
    h = h + bl_ref[...]
    h = h + lax.dot_general(x_ref[...], wr_ref[...], (((1,), (1,)), ((), ())),
                            preferred_element_type=jnp.float32)
    out_ref[...] = jnp.maximum(h, 0.0)


def _tc_layer0(sums2, cnt_col, x, W_l0, b_l0, W_r0):
    BN = 2000
    return pl.pallas_call(
        _tc_layer0_body,
        grid=(N1 // BN,),
        in_specs=[
            pl.BlockSpec((NC, BN, DH), lambda i: (0, i, 0)),
            pl.BlockSpec((BN, 1), lambda i: (i, 0)),
            pl.BlockSpec((BN, D), lambda i: (i, 0)),
            pl.BlockSpec((D, D), lambda i: (0, 0)),
            pl.BlockSpec((1, D), lambda i: (0, 0)),
            pl.BlockSpec((D, D), lambda i: (0, 0)),
        ],
        out_specs=pl.BlockSpec((BN, D), lambda i: (i, 0)),
        out_shape=jax.ShapeDtypeStruct((N1, D), jnp.float32),
        name="tc_sage_layer0",
    )(sums2, cnt_col, x, W_l0, b_l0, W_r0)


def _tc_layer1_body(sum_ref, cntc_ref, h_ref, wl_ref, bl_ref, wr_ref,
                    wp_ref, bp_ref, out_ref):
    ssum = sum_ref[0] + sum_ref[1]
    cnt = cntc_ref[:, 0:1] + cntc_ref[:, 1:2]
    mean = ssum / jnp.maximum(cnt, 1.0)
    h1 = lax.dot_general(mean, wl_ref[...], (((1,), (1,)), ((), ())),
                         preferred_element_type=jnp.float32)
    h1 = h1 + bl_ref[...]
    h1 = h1 + lax.dot_general(h_ref[...], wr_ref[...], (((1,), (1,)), ((), ())),
                              preferred_element_type=jnp.float32)
    o = lax.dot_general(h1, wp_ref[...], (((1,), (1,)), ((), ())),
                        preferred_element_type=jnp.float32)
    o = o + bp_ref[...]
    m = jnp.max(o, axis=1, keepdims=True)
    lse = jnp.log(jnp.sum(jnp.exp(o - m), axis=1, keepdims=True)) + m
    out_ref[...] = o - lse


def _tc_layer1(sums, cnts_col, h, W_l1, b_l1, W_r1, W_post, b_post):
    return pl.pallas_call(
        _tc_layer1_body,
        grid=(1,),
        in_specs=[
            pl.BlockSpec((NC, N2, D), lambda i: (0, 0, 0)),
            pl.BlockSpec((N2, NC), lambda i: (0, 0)),
            pl.BlockSpec((N2, D), lambda i: (0, 0)),
            pl.BlockSpec((D, D), lambda i: (0, 0)),
            pl.BlockSpec((1, D), lambda i: (0, 0)),
            pl.BlockSpec((D, D), lambda i: (0, 0)),
            pl.BlockSpec((D, D), lambda i: (0, 0)),
            pl.BlockSpec((1, D), lambda i: (0, 0)),
        ],
        out_specs=pl.BlockSpec((N2, D), lambda i: (0, 0)),
        out_shape=jax.ShapeDtypeStruct((N2, D), jnp.float32),
        name="tc_sage_layer1",
    )(sums, cnts_col, h, W_l1, b_l1, W_r1, W_post, b_post)


def kernel(x, edge_index_l0, edge_index_l1, W_l0, b_l0, W_r0,
           W_l1, b_l1, W_r1, W_post, b_post):
    src0 = edge_index_l0[0].astype(jnp.int32)
    dst0 = edge_index_l0[1].astype(jnp.int32)
    src1 = edge_index_l1[0].astype(jnp.int32)
    dst1 = edge_index_l1[1].astype(jnp.int32)

    npad = E1_PAD - E1
    src0 = jnp.concatenate([src0, jnp.zeros((npad,), jnp.int32)])
    dst0 = jnp.concatenate([dst0, jnp.full((npad,), SENT0, jnp.int32)])
    src0 = src0.reshape(NS, 2, KH0, CHUNK)
    dst0 = dst0.reshape(NS, 2, KH0, CHUNK)
    src1 = src1.reshape(NW, K1, CHUNK)
    dst1 = dst1.reshape(NW, K1, CHUNK)

    x2 = x.reshape(2 * N, DH)
    sums0, cnt0 = _sc_agg0(x2, src0, dst0)
    sums0 = sums0.reshape(NC, ACC0, DH)
    cnt0_col = cnt0.reshape(ACC0, 1)
    h = _tc_layer0(sums0, cnt0_col, x, W_l0, b_l0.reshape(1, D), W_r0)

    sums1, cnts1 = _sc_agg1(h, src1, dst1)
    sums1 = sums1.reshape(NC, ACC1, D)
    cnts1_col = cnts1.T
    return _tc_layer1(sums1, cnts1_col, h, W_l1, b_l1.reshape(1, D), W_r1,
                      W_post, b_post.reshape(1, D))
